# baseline (speedup 1.0000x reference)
import functools

import jax
import jax.numpy as jnp
from jax import lax
from jax.experimental import pallas as pl
from jax.experimental.pallas import tpu as pltpu
from jax.experimental.pallas import tpu_sc as plsc


_DIAG_BLOCKS = 10
_DIAG_PAD = 1024


def _diag_body(x_ref, p_ref, b_ref, out_ref):
    z = jnp.dot(x_ref[...], p_ref[...], preferred_element_type=jnp.float32)
    sig = jax.nn.sigmoid(z + b_ref[...]).T
    pad = jnp.zeros((1, _DIAG_PAD - sig.shape[1]), jnp.float32)
    out_ref[...] = jnp.concatenate([sig, pad], axis=1)


@functools.cache
def _diag_call(n, d):
    rows = n // _DIAG_BLOCKS
    return pl.pallas_call(
        _diag_body,
        grid=(_DIAG_BLOCKS,),
        in_specs=[
            pl.BlockSpec((rows, d), lambda i: (i, 0)),
            pl.BlockSpec((d, 1), lambda i: (0, 0)),
            pl.BlockSpec((1, 1), lambda i: (0, 0)),
        ],
        out_specs=pl.BlockSpec((1, _DIAG_PAD), lambda i: (0, i)),
        out_shape=jax.ShapeDtypeStruct((1, _DIAG_BLOCKS * _DIAG_PAD), jnp.float32),
    )


_NUM_CORES = 2
_NUM_SUBCORES = 16
_LANES = 16


@functools.cache
def _gather_call(n, e):
    nw = _NUM_CORES * _NUM_SUBCORES
    lanes = _LANES
    main = (e // nw) // 128 * 128
    extra = e - nw * main
    assert main % lanes == 0 and extra % lanes == 0 and extra >= 0
    nvec_main = main // lanes
    nvec_extra = extra // lanes
    mesh = plsc.VectorSubcoreMesh(
        core_axis_name="c", subcore_axis_name="s",
        num_cores=_NUM_CORES, num_subcores=_NUM_SUBCORES,
    )
    n_pad = _DIAG_BLOCKS * _DIAG_PAD
    rows = n // _DIAG_BLOCKS
    pad_step = _DIAG_PAD - rows
    recip = (1 << 24) // rows + 1

    @functools.partial(
        pl.kernel,
        out_type=(
            jax.ShapeDtypeStruct((2, e), jnp.int32),
            jax.ShapeDtypeStruct((e,), jnp.float32),
        ),
        mesh=mesh,
        compiler_params=pltpu.CompilerParams(needs_layout_passes=False),
        scratch_types=[
            pltpu.VMEM((n_pad,), jnp.float32),
            pltpu.VMEM((2, main + extra), jnp.int32),
            pltpu.VMEM((main + extra,), jnp.float32),
            pltpu.VMEM((main + extra,), jnp.float32),
            pltpu.SemaphoreType.DMA,
            pltpu.SemaphoreType.DMA,
        ],
    )
    def gather_k(diag_hbm, ei_hbm, attr_hbm, ei_out, val_out,
                 diag_v, ei_v, attr_v, val_v, sem_in, sem_out):
        wid = lax.axis_index("s") * _NUM_CORES + lax.axis_index("c")
        base = wid * main
        is_last = wid == nw - 1

        cp_diag = pltpu.make_async_copy(diag_hbm.at[0, :], diag_v, sem_in)
        cp_ei = pltpu.make_async_copy(
            ei_hbm.at[:, pl.ds(base, main)], ei_v.at[:, pl.ds(0, main)], sem_in)
        cp_attr = pltpu.make_async_copy(
            attr_hbm.at[pl.ds(base, main)], attr_v.at[pl.ds(0, main)], sem_in)
        cp_diag.start()
        cp_ei.start()
        cp_attr.start()

        ebase = nw * main
        cp_ei_x = pltpu.make_async_copy(
            ei_hbm.at[:, pl.ds(ebase, extra)],
            ei_v.at[:, pl.ds(main, extra)], sem_in)
        cp_attr_x = pltpu.make_async_copy(
            attr_hbm.at[pl.ds(ebase, extra)],
            attr_v.at[pl.ds(main, extra)], sem_in)

        @pl.when(is_last)
        def _():
            cp_ei_x.start()
            cp_attr_x.start()

        cp_diag.wait()
        cp_ei.wait()
        cp_attr.wait()

        @pl.when(is_last)
        def _():
            cp_ei_x.wait()
            cp_attr_x.wait()

        cp_po = pltpu.make_async_copy(
            ei_v.at[:, pl.ds(0, main)], ei_out.at[:, pl.ds(base, main)], sem_out)
        cp_po.start()
        cp_po_x = pltpu.make_async_copy(
            ei_v.at[:, pl.ds(main, extra)], ei_out.at[:, pl.ds(ebase, extra)],
            sem_out)

        @pl.when(is_last)
        def _():
            cp_po_x.start()

        def body(i, carry):
            s = pl.ds(i * lanes, lanes)
            idx = ei_v[1, s]
            blk = lax.shift_right_logical(idx * recip, 24)
            loc = idx + blk * pad_step
            vals = plsc.load_gather(diag_v, [loc])
            val_v[s] = vals * attr_v[s]
            return carry

        lax.fori_loop(0, nvec_main, body, 0, unroll=8)

        @pl.when(is_last)
        def _():
            lax.fori_loop(nvec_main, nvec_main + nvec_extra, body, 0, unroll=8)

        cp_val = pltpu.make_async_copy(
            val_v.at[pl.ds(0, main)], val_out.at[pl.ds(base, main)], sem_out)
        cp_val.start()
        cp_val_x = pltpu.make_async_copy(
            val_v.at[pl.ds(main, extra)], val_out.at[pl.ds(ebase, extra)],
            sem_out)

        @pl.when(is_last)
        def _():
            cp_val_x.start()

        cp_po.wait()
        cp_val.wait()

        @pl.when(is_last)
        def _():
            cp_po_x.wait()
            cp_val_x.wait()

    return gather_k


def kernel(x, edge_index, edge_attr, p, b):
    n, d = x.shape
    e = edge_attr.shape[0]
    diag2d = _diag_call(n, d)(x, p, b.reshape(1, 1))
    ei_out, adj_val = _gather_call(n, e)(diag2d, edge_index, edge_attr)
    return (ei_out, adj_val)


# speedup vs baseline: 1.1949x; 1.1949x over previous
import functools

import jax
import jax.numpy as jnp
from jax import lax
from jax.experimental import pallas as pl
from jax.experimental.pallas import tpu as pltpu
from jax.experimental.pallas import tpu_sc as plsc


def _diag_body(x_ref, p_ref, b_ref, out_ref):
    z = jnp.dot(x_ref[...], p_ref[...], preferred_element_type=jnp.float32)
    out_ref[...] = jax.nn.sigmoid(z + b_ref[...]).T


@functools.cache
def _diag_call(n, d):
    return pl.pallas_call(
        _diag_body,
        out_shape=jax.ShapeDtypeStruct((1, n), jnp.float32),
    )


_NUM_CORES = 2
_NUM_SUBCORES = 16
_LANES = 16


@functools.cache
def _gather_call(n, e):
    nw = _NUM_CORES * _NUM_SUBCORES
    lanes = _LANES
    main = (e // nw) // 128 * 128
    extra = e - nw * main
    assert main % lanes == 0 and extra % lanes == 0 and extra >= 0
    nvec_main = main // lanes
    nvec_extra = extra // lanes
    mesh = plsc.VectorSubcoreMesh(
        core_axis_name="c", subcore_axis_name="s",
        num_cores=_NUM_CORES, num_subcores=_NUM_SUBCORES,
    )

    @functools.partial(
        pl.kernel,
        out_type=(
            jax.ShapeDtypeStruct((2, e), jnp.int32),
            jax.ShapeDtypeStruct((e,), jnp.float32),
        ),
        mesh=mesh,
        compiler_params=pltpu.CompilerParams(needs_layout_passes=False),
        scratch_types=[
            pltpu.VMEM((n,), jnp.float32),
            pltpu.VMEM((2, main + extra), jnp.int32),
            pltpu.VMEM((main + extra,), jnp.float32),
            pltpu.VMEM((main + extra,), jnp.float32),
            pltpu.SemaphoreType.DMA,
            pltpu.SemaphoreType.DMA,
        ],
    )
    def gather_k(diag_hbm, ei_hbm, attr_hbm, ei_out, val_out,
                 diag_v, ei_v, attr_v, val_v, sem_in, sem_out):
        wid = lax.axis_index("s") * _NUM_CORES + lax.axis_index("c")
        base = wid * main
        is_last = wid == nw - 1

        half = main // 2
        cp_diag = pltpu.make_async_copy(diag_hbm.at[0, :], diag_v, sem_in)
        cp_ei = pltpu.make_async_copy(
            ei_hbm.at[:, pl.ds(base, half)], ei_v.at[:, pl.ds(0, half)], sem_in)
        cp_attr = pltpu.make_async_copy(
            attr_hbm.at[pl.ds(base, half)], attr_v.at[pl.ds(0, half)], sem_in)
        cp_ei2 = pltpu.make_async_copy(
            ei_hbm.at[:, pl.ds(base + half, half)],
            ei_v.at[:, pl.ds(half, half)], sem_in)
        cp_attr2 = pltpu.make_async_copy(
            attr_hbm.at[pl.ds(base + half, half)],
            attr_v.at[pl.ds(half, half)], sem_in)
        cp_diag.start()
        cp_ei.start()
        cp_attr.start()
        cp_ei2.start()
        cp_attr2.start()

        ebase = nw * main
        cp_ei_x = pltpu.make_async_copy(
            ei_hbm.at[:, pl.ds(ebase, extra)],
            ei_v.at[:, pl.ds(main, extra)], sem_in)
        cp_attr_x = pltpu.make_async_copy(
            attr_hbm.at[pl.ds(ebase, extra)],
            attr_v.at[pl.ds(main, extra)], sem_in)

        @pl.when(is_last)
        def _():
            cp_ei_x.start()
            cp_attr_x.start()

        def body(i, carry):
            s = pl.ds(i * lanes, lanes)
            idx = ei_v[1, s]
            vals = plsc.load_gather(diag_v, [idx])
            val_v[s] = vals * attr_v[s]
            return carry

        cp_diag.wait()
        cp_ei.wait()
        cp_attr.wait()
        lax.fori_loop(0, nvec_main // 2, body, 0, unroll=8)
        cp_val1 = pltpu.make_async_copy(
            val_v.at[pl.ds(0, half)], val_out.at[pl.ds(base, half)], sem_out)
        cp_val1.start()
        cp_po = pltpu.make_async_copy(
            ei_v.at[:, pl.ds(0, half)], ei_out.at[:, pl.ds(base, half)], sem_out)
        cp_po.start()

        cp_ei2.wait()
        cp_attr2.wait()

        @pl.when(is_last)
        def _():
            cp_ei_x.wait()
            cp_attr_x.wait()

        cp_po2 = pltpu.make_async_copy(
            ei_v.at[:, pl.ds(half, half)],
            ei_out.at[:, pl.ds(base + half, half)], sem_out)
        cp_po2.start()
        cp_po_x = pltpu.make_async_copy(
            ei_v.at[:, pl.ds(main, extra)], ei_out.at[:, pl.ds(ebase, extra)],
            sem_out)

        @pl.when(is_last)
        def _():
            cp_po_x.start()

        lax.fori_loop(nvec_main // 2, nvec_main, body, 0, unroll=8)

        @pl.when(is_last)
        def _():
            lax.fori_loop(nvec_main, nvec_main + nvec_extra, body, 0, unroll=8)

        cp_val = pltpu.make_async_copy(
            val_v.at[pl.ds(half, half)],
            val_out.at[pl.ds(base + half, half)], sem_out)
        cp_val.start()
        cp_val_x = pltpu.make_async_copy(
            val_v.at[pl.ds(main, extra)], val_out.at[pl.ds(ebase, extra)],
            sem_out)

        @pl.when(is_last)
        def _():
            cp_val_x.start()

        cp_po.wait()
        cp_po2.wait()
        cp_val1.wait()
        cp_val.wait()

        @pl.when(is_last)
        def _():
            cp_po_x.wait()
            cp_val_x.wait()

    return gather_k


def kernel(x, edge_index, edge_attr, p, b):
    n, d = x.shape
    e = edge_attr.shape[0]
    diag2d = _diag_call(n, d)(x, p, b.reshape(1, 1))
    ei_out, adj_val = _gather_call(n, e)(diag2d, edge_index, edge_attr)
    return (ei_out, adj_val)
